# baseline (speedup 1.0000x reference)
import jax, jax.numpy as jnp
from jax.experimental import pallas as pl
from jax.experimental.pallas import tpu as pltpu

_N = 10000
_B = 5000
_IN = ((0, 2000), (2000, 3000), (5000, 5000))


def _gcn_kernel(x_hbm, w_ref, b_ref, o_ref, x_v, sems):
    i = pl.program_id(0)

    @pl.when(i == 0)
    def _first():
        for c, (base, size) in enumerate(_IN):
            rows = pl.ds(base, size)
            pltpu.make_async_copy(
                x_hbm.at[rows, :], x_v.at[rows, :], sems.at[c]
            ).start()
        w = w_ref[...]
        b = b_ref[...]
        for c in (0, 1):
            base, size = _IN[c]
            rows = pl.ds(base, size)
            pltpu.make_async_copy(
                x_hbm.at[rows, :], x_v.at[rows, :], sems.at[c]
            ).wait()
            o_ref[pl.ds(base, size), :] = (
                jnp.dot(x_v[rows, :], w, preferred_element_type=jnp.float32)
                + b
            )

    @pl.when(i == 1)
    def _second():
        base, size = _IN[2]
        rows = pl.ds(base, size)
        pltpu.make_async_copy(
            x_hbm.at[rows, :], x_v.at[rows, :], sems.at[2]
        ).wait()
        o_ref[...] = (
            jnp.dot(x_v[rows, :], w_ref[...], preferred_element_type=jnp.float32)
            + b_ref[...]
        )


def kernel(An, X, weight, bias):
    del An
    n, d = X.shape
    units = weight.shape[1]
    bias2d = bias.reshape(1, units)
    return pl.pallas_call(
        _gcn_kernel,
        grid=(n // _B,),
        in_specs=[
            pl.BlockSpec(memory_space=pltpu.MemorySpace.HBM),
            pl.BlockSpec(memory_space=pltpu.MemorySpace.VMEM),
            pl.BlockSpec(memory_space=pltpu.MemorySpace.VMEM),
        ],
        out_specs=pl.BlockSpec((_B, units), lambda i: (i, 0)),
        out_shape=jax.ShapeDtypeStruct((n, units), jnp.float32),
        scratch_shapes=[
            pltpu.MemorySpace.VMEM((n, d), jnp.float32),
            pltpu.SemaphoreType.DMA((len(_IN),)),
        ],
    )(X, weight, bias2d)


# speedup vs baseline: 1.1405x; 1.1405x over previous
import jax
import jax.numpy as jnp
from jax.experimental import pallas as pl
from jax.experimental.pallas import tpu as pltpu

_BLOCK_ROWS = 5000


def _gcn_kernel(x_ref, w_ref, b_ref, o_ref):
    o_ref[...] = (
        jnp.dot(x_ref[...], w_ref[...], preferred_element_type=jnp.float32)
        + b_ref[...]
    )


def kernel(An, X, weight, bias):
    del An
    n, d = X.shape
    units = weight.shape[1]
    bias2d = bias.reshape(1, units)
    return pl.pallas_call(
        _gcn_kernel,
        grid=(n // _BLOCK_ROWS,),
        in_specs=[
            pl.BlockSpec((_BLOCK_ROWS, d), lambda i: (i, 0)),
            pl.BlockSpec((d, units), lambda i: (0, 0)),
            pl.BlockSpec((1, units), lambda i: (0, 0)),
        ],
        out_specs=pl.BlockSpec((_BLOCK_ROWS, units), lambda i: (i, 0)),
        out_shape=jax.ShapeDtypeStruct((n, units), jnp.float32),
        compiler_params=pltpu.CompilerParams(
            dimension_semantics=("arbitrary",),
        ),
    )(X, weight, bias2d)


# speedup vs baseline: 1.1511x; 1.0093x over previous
import jax
import jax.numpy as jnp
from jax.experimental import pallas as pl
from jax.experimental.pallas import tpu as pltpu

_BLOCK_ROWS = 5000


def _gcn_kernel(x_ref, w_ref, b_ref, o_ref):
    o_ref[...] = (
        jnp.dot(x_ref[...], w_ref[...], preferred_element_type=jnp.float32)
        + b_ref[...]
    )


def kernel(An, X, weight, bias):
    del An
    n, d = X.shape
    units = weight.shape[1]
    bias2d = bias.reshape(1, units)
    return pl.pallas_call(
        _gcn_kernel,
        grid=(n // _BLOCK_ROWS,),
        in_specs=[
            pl.BlockSpec((_BLOCK_ROWS, d), lambda i: (i, 0)),
            pl.BlockSpec((d, units), lambda i: (0, 0)),
            pl.BlockSpec((1, units), lambda i: (0, 0)),
        ],
        out_specs=pl.BlockSpec((_BLOCK_ROWS, units), lambda i: (i, 0)),
        out_shape=jax.ShapeDtypeStruct((n, units), jnp.float32),
        compiler_params=pltpu.CompilerParams(
            dimension_semantics=("parallel",),
        ),
    )(X, weight, bias2d)
